# baseline (speedup 1.0000x reference)
import functools

import jax
import jax.numpy as jnp
from jax import lax
from jax.experimental import pallas as pl
from jax.experimental.pallas import tpu as pltpu
from jax.experimental.pallas import tpu_sc as plsc

_LEN = 2048
_C = 128
_B = 32
_NC = 2
_NS = 16
_PB = 128
_BH = _B // 2
_R = 6
_GLAG = 4


def _pe_table():
    i = jnp.arange(_LEN, dtype=jnp.float32)[:, None]
    j = jnp.arange(_C, dtype=jnp.float32)[None, :]
    val = i / jnp.power(10000.0, j / float(_C))
    even = (jnp.arange(_C)[None, :] % 2) == 0
    return jnp.where(even, jnp.sin(val), jnp.cos(val))


@functools.partial(
    pl.kernel,
    out_type=jax.ShapeDtypeStruct((_B, _LEN, _C), jnp.float32),
    mesh=plsc.VectorSubcoreMesh(core_axis_name="c", subcore_axis_name="s"),
    scratch_types=[
        pltpu.VMEM((_BH, _PB), jnp.int32),
        pltpu.VMEM((4, _PB, _C), jnp.float32),
        pltpu.VMEM_SHARED((_NS, _PB, _C), jnp.float32),
        pltpu.VMEM_SHARED((_NS, 2, _PB, _C), jnp.float32),
        pltpu.SemaphoreType.DMA((4,)),
        pltpu.SemaphoreType.DMA((4,)),
        pltpu.SemaphoreType.DMA((4,)),
        pltpu.SemaphoreType.DMA((2,)),
    ],
)
def _embed_sc(x_hbm, w_hbm, pe_hbm, out_hbm, idx_v, buf_v, pe_sh, st_sh,
              sems_s, sems_g, sems_x, sems_h):
    c = lax.axis_index("c")
    s = lax.axis_index("s")
    p0 = s * _PB
    b0 = c * _BH
    pltpu.sync_copy(x_hbm.at[pl.ds(b0, _BH), pl.ds(p0, _PB)], idx_v)
    pltpu.sync_copy(pe_hbm.at[pl.ds(p0, _PB)], buf_v.at[0])
    pltpu.sync_copy(buf_v.at[0], pe_sh.at[s])
    pe_slot = pe_sh.at[s]

    seed = [None] * 4
    gat = [None] * 4
    cross = [None] * 4
    flush = [None] * 2

    seed[0] = pltpu.async_copy(pe_slot, buf_v.at[0], sems_s.at[0])
    for t in range(_BH + 5):
        if t < _BH:
            sl = t % 4
            seed[sl].wait()
            gat[sl] = pltpu.async_copy(
                w_hbm.at[idx_v.at[t]], buf_v.at[sl], sems_g.at[sl], add=True)
        bc = t - 3
        if 0 <= bc < _BH:
            sl = bc % 4
            gat[sl].wait()
            if bc >= 2:
                flush[bc % 2].wait()
            cross[sl] = pltpu.async_copy(
                buf_v.at[sl], st_sh.at[s, bc % 2], sems_x.at[sl])
        bs = t + 1
        if bs < _BH:
            sl = bs % 4
            if bs >= 4:
                cross[sl].wait()
            seed[sl] = pltpu.async_copy(pe_slot, buf_v.at[sl], sems_s.at[sl])
        bf = t - 4
        if 0 <= bf < _BH:
            if bf + 4 >= _BH:
                cross[bf % 4].wait()
            flush[bf % 2] = pltpu.async_copy(
                st_sh.at[s, bf % 2], out_hbm.at[b0 + bf, pl.ds(p0, _PB)],
                sems_h.at[bf % 2])
    flush[0].wait()
    flush[1].wait()


def kernel(x, W):
    pe = _pe_table()
    return _embed_sc(x.astype(jnp.int32), W, pe)


# speedup vs baseline: 1.2961x; 1.2961x over previous
import functools

import jax
import jax.numpy as jnp
from jax import lax
from jax.experimental import pallas as pl
from jax.experimental.pallas import tpu as pltpu
from jax.experimental.pallas import tpu_sc as plsc

_LEN = 2048
_C = 128
_B = 32
_NC = 2
_NS = 16
_PB = 128
_BH = _B // 2
_R = 6
_GLAG = 4


def _pe_table():
    i = jnp.arange(_LEN, dtype=jnp.float32)[:, None]
    j = jnp.arange(_C, dtype=jnp.float32)[None, :]
    val = i / jnp.power(10000.0, j / float(_C))
    even = (jnp.arange(_C)[None, :] % 2) == 0
    return jnp.where(even, jnp.sin(val), jnp.cos(val))


@functools.partial(
    pl.kernel,
    out_type=jax.ShapeDtypeStruct((_B, _LEN, _C), jnp.float32),
    mesh=plsc.VectorSubcoreMesh(core_axis_name="c", subcore_axis_name="s"),
    scratch_types=[
        pltpu.VMEM((_BH, _PB), jnp.int32),
        pltpu.VMEM((_R, _PB, _C), jnp.float32),
        pltpu.VMEM_SHARED((_NS, _PB, _C), jnp.float32),
        pltpu.SemaphoreType.DMA((_R,)),
        pltpu.SemaphoreType.DMA((_R,)),
        pltpu.SemaphoreType.DMA((_R,)),
    ],
)
def _embed_sc(x_hbm, w_hbm, pe_hbm, out_hbm, idx_v, buf_v, pe_sh,
              sems_s, sems_g, sems_o):
    c = lax.axis_index("c")
    s = lax.axis_index("s")
    p0 = s * _PB
    b0 = c * _BH
    pltpu.sync_copy(x_hbm.at[pl.ds(b0, _BH), pl.ds(p0, _PB)], idx_v)
    pltpu.sync_copy(pe_hbm.at[pl.ds(p0, _PB)], buf_v.at[0])
    pltpu.sync_copy(buf_v.at[0], pe_sh.at[s])
    pe_slot = pe_sh.at[s]

    seed = [None] * _R
    gat = [None] * _R
    outw = [None] * _R

    seed[0] = pltpu.async_copy(pe_slot, buf_v.at[0], sems_s.at[0])
    for t in range(_BH + _GLAG + 1):
        bw = t - _GLAG
        if 0 <= bw < _BH:
            sl = bw % _R
            gat[sl].wait()
            outw[sl] = pltpu.async_copy(
                buf_v.at[sl], out_hbm.at[b0 + bw, pl.ds(p0, _PB)],
                sems_o.at[sl])
        bs = t + 1
        if bs < _BH:
            sl = bs % _R
            if bs >= _R:
                outw[sl].wait()
            seed[sl] = pltpu.async_copy(pe_slot, buf_v.at[sl], sems_s.at[sl])
        if t < _BH:
            sl = t % _R
            seed[sl].wait()
            gat[sl] = pltpu.async_copy(
                w_hbm.at[idx_v.at[t]], buf_v.at[sl], sems_g.at[sl], add=True)
    for b in range(_BH - _R, _BH):
        outw[b % _R].wait()


def kernel(x, W):
    pe = _pe_table()
    return _embed_sc(x.astype(jnp.int32), W, pe)
